# baseline (speedup 1.0000x reference)
import functools

import jax
import jax.numpy as jnp
from jax import lax
from jax.experimental import pallas as pl
from jax.experimental.pallas import tpu as pltpu
from jax.experimental.pallas import tpu_sc as plsc

E = 3200000
N_NODES = 100000
NPAD = 102400
NC, NS = 2, 16
NW = NC * NS
CH = 1024
NCHUNK = E // CH
CHUNKS_PER_TILE = NCHUNK // NW + 1


def _silu(x):
    return x * jax.nn.sigmoid(x)


def _edge_mlp_body(x_ref, w1_ref, b1_ref, w2_ref, b2_ref, o_ref):
    x = x_ref[...].astype(jnp.bfloat16)
    w1 = w1_ref[...].astype(jnp.bfloat16)
    w2 = w2_ref[...].astype(jnp.bfloat16)
    y = _silu(jnp.dot(x, w1, preferred_element_type=jnp.float32) + b1_ref[...])
    z = _silu(jnp.dot(y.astype(jnp.bfloat16), w2, preferred_element_type=jnp.float32) + b2_ref[...])
    o_ref[...] = z


def _edge_mlp(x2, bd1, b1t, bd2, b2t):
    rows = x2.shape[0]
    blk = 4000
    grid = rows // blk
    return pl.pallas_call(
        _edge_mlp_body,
        grid=(grid,),
        in_specs=[
            pl.BlockSpec((blk, 128), lambda i: (i, 0)),
            pl.BlockSpec((128, 128), lambda i: (0, 0)),
            pl.BlockSpec((1, 128), lambda i: (0, 0)),
            pl.BlockSpec((128, 128), lambda i: (0, 0)),
            pl.BlockSpec((1, 128), lambda i: (0, 0)),
        ],
        out_specs=pl.BlockSpec((blk, 128), lambda i: (i, 0)),
        out_shape=jax.ShapeDtypeStruct((rows, 128), jnp.float32),
    )(x2, bd1, b1t, bd2, b2t)


def _sc_scatter_body(e_hbm, row2d_hbm, out_hbm, agg_sh, ebuf, ibuf):
    c = lax.axis_index("c")
    s = lax.axis_index("s")
    w = s * NC + c

    def _zrow(i, carry):
        ebuf[i] = jnp.zeros((16,), jnp.float32)
        return carry

    lax.fori_loop(0, CH, _zrow, 0)
    base = s * (NPAD // NS)
    for z in range(6):
        pltpu.sync_copy(ebuf, agg_sh.at[pl.ds(base + z * CH, CH)])
    pltpu.sync_copy(ebuf.at[pl.ds(0, 256)], agg_sh.at[pl.ds(base + 6 * CH, 256)])
    plsc.subcore_barrier()

    def _do_chunk(start_edge):
        start_edge = pl.multiple_of(start_edge, CH)
        pltpu.sync_copy(e_hbm.at[pl.ds(start_edge, CH)], ebuf)
        pltpu.sync_copy(
            row2d_hbm.at[pl.ds(pl.multiple_of(start_edge // 128, 8), CH // 128)],
            ibuf,
        )
        for sub in range(CH // 128):
            pltpu.sync_copy(
                ebuf.at[pl.ds(sub * 128, 128)],
                agg_sh.at[ibuf.at[sub]],
                add=True,
            )

    def _loop(k, carry):
        j = w + NW * k

        @pl.when(j < NCHUNK)
        def _():
            _do_chunk(j * CH)

        return carry

    lax.fori_loop(0, CHUNKS_PER_TILE, _loop, 0)
    plsc.subcore_barrier()

    pltpu.sync_copy(
        agg_sh.at[pl.ds(base, NPAD // NS)],
        out_hbm.at[c].at[pl.ds(base, NPAD // NS)],
    )


def _sc_scatter(e, row2d):
    mesh = plsc.VectorSubcoreMesh(core_axis_name="c", subcore_axis_name="s")
    f = pl.kernel(
        _sc_scatter_body,
        out_type=jax.ShapeDtypeStruct((NC, NPAD, 16), jnp.float32),
        mesh=mesh,
        scratch_types=[
            pltpu.VMEM_SHARED((NPAD, 16), jnp.float32),
            pltpu.VMEM((CH, 16), jnp.float32),
            pltpu.VMEM((CH // 128, 128), jnp.int32),
        ],
        compiler_params=pltpu.CompilerParams(use_tc_tiling_on_sc=False),
    )
    return f(e, row2d)


def _node_mlp_body(p_ref, w3_ref, b3_ref, w4_ref, b4_ref, o_ref):
    x = p_ref[0] + p_ref[1]
    y = _silu(jnp.dot(x.astype(jnp.bfloat16), w3_ref[...].astype(jnp.bfloat16),
                      preferred_element_type=jnp.float32) + b3_ref[...])
    o_ref[...] = jnp.dot(y.astype(jnp.bfloat16), w4_ref[...].astype(jnp.bfloat16),
                         preferred_element_type=jnp.float32) + b4_ref[...]


def _node_mlp(p, w3, b3t, w4, b4t):
    blk = 10000
    grid = N_NODES // blk
    return pl.pallas_call(
        _node_mlp_body,
        grid=(grid,),
        in_specs=[
            pl.BlockSpec((NC, blk, 16), lambda i: (0, i, 0)),
            pl.BlockSpec((16, 16), lambda i: (0, 0)),
            pl.BlockSpec((1, 16), lambda i: (0, 0)),
            pl.BlockSpec((16, 128), lambda i: (0, 0)),
            pl.BlockSpec((1, 128), lambda i: (0, 0)),
        ],
        out_specs=pl.BlockSpec((blk, 128), lambda i: (i, 0)),
        out_shape=jax.ShapeDtypeStruct((N_NODES, 128), jnp.float32),
    )(p, w3, b3t, w4, b4t)


def kernel(edge_index, edge_attr, W1, b1, W2, b2, W3, b3, W4, b4):
    eye8 = jnp.eye(8, dtype=jnp.float32)
    bd1 = jnp.kron(eye8, W1)
    bd2 = jnp.kron(eye8, W2)
    b1t = jnp.tile(b1, 8)[None, :]
    b2t = jnp.tile(b2, 8)[None, :]

    x2 = edge_attr.reshape(E // 8, 128)
    e2 = _edge_mlp(x2, bd1, b1t, bd2, b2t)

    row2d = edge_index[0].reshape(E // 128, 128)
    e = e2.reshape(E, 16)
    p = _sc_scatter(e, row2d)

    return _node_mlp(p, W3, b3[None, :], W4, b4[None, :])


# speedup vs baseline: 1.9268x; 1.9268x over previous
import functools

import jax
import jax.numpy as jnp
from jax import lax
from jax.experimental import pallas as pl
from jax.experimental.pallas import tpu as pltpu
from jax.experimental.pallas import tpu_sc as plsc

E = 3200000
N_NODES = 100000
NPAD = 102400
NC, NS = 2, 16
NW = NC * NS
CH = 1024
NCHUNK = E // CH
CHUNKS_PER_TILE = NCHUNK // NW + 1
BLK = 25600
G = BLK // 8


def _silu(x):
    return x * jax.nn.sigmoid(x)


def _edge_mlp_body(x_ref, w1_ref, b1_ref, w2_ref, b2_ref, o_ref):
    xb = x_ref[...]
    xt = jnp.concatenate([xb[:, s * G:(s + 1) * G] for s in range(8)], axis=0)
    xt = xt.astype(jnp.bfloat16)
    w1 = w1_ref[...].astype(jnp.bfloat16)
    w2 = w2_ref[...].astype(jnp.bfloat16)
    y = lax.dot_general(xt, w1, (((0,), (0,)), ((), ())),
                        preferred_element_type=jnp.float32)
    y = _silu(y + b1_ref[...])
    z = jnp.dot(y.astype(jnp.bfloat16), w2,
                preferred_element_type=jnp.float32) + b2_ref[...]
    o_ref[...] = _silu(z)


def _edge_mlp(xT, bd1, b1t, bd2, b2t):
    grid = E // BLK
    return pl.pallas_call(
        _edge_mlp_body,
        grid=(grid,),
        in_specs=[
            pl.BlockSpec((16, BLK), lambda i: (0, i)),
            pl.BlockSpec((128, 128), lambda i: (0, 0)),
            pl.BlockSpec((1, 128), lambda i: (0, 0)),
            pl.BlockSpec((128, 128), lambda i: (0, 0)),
            pl.BlockSpec((1, 128), lambda i: (0, 0)),
        ],
        out_specs=pl.BlockSpec((G, 128), lambda i: (i, 0)),
        out_shape=jax.ShapeDtypeStruct((E // 8, 128), jnp.float32),
    )(xT, bd1, b1t, bd2, b2t)


def _sc_scatter_body(e_hbm, row2d_hbm, out_hbm, agg_sh, ebuf, ibuf):
    c = lax.axis_index("c")
    s = lax.axis_index("s")
    w = s * NC + c

    def _zrow(i, carry):
        ebuf[i] = jnp.zeros((16,), jnp.float32)
        return carry

    lax.fori_loop(0, CH, _zrow, 0)
    base = s * (NPAD // NS)
    for z in range(6):
        pltpu.sync_copy(ebuf, agg_sh.at[pl.ds(base + z * CH, CH)])
    pltpu.sync_copy(ebuf.at[pl.ds(0, 256)], agg_sh.at[pl.ds(base + 6 * CH, 256)])
    plsc.subcore_barrier()

    def _do_chunk(start_edge):
        start_edge = pl.multiple_of(start_edge, CH)
        pltpu.sync_copy(e_hbm.at[pl.ds(start_edge, CH)], ebuf)
        pltpu.sync_copy(
            row2d_hbm.at[pl.ds(pl.multiple_of(start_edge // 128, 8), CH // 128)],
            ibuf,
        )
        for sub in range(CH // 128):
            pltpu.sync_copy(
                ebuf.at[pl.ds(sub * 128, 128)],
                agg_sh.at[ibuf.at[sub]],
                add=True,
            )

    def _loop(k, carry):
        j = w + NW * k

        @pl.when(j < NCHUNK)
        def _():
            _do_chunk(j * CH)

        return carry

    lax.fori_loop(0, CHUNKS_PER_TILE, _loop, 0)
    plsc.subcore_barrier()

    pltpu.sync_copy(
        agg_sh.at[pl.ds(base, NPAD // NS)],
        out_hbm.at[c].at[pl.ds(base, NPAD // NS)],
    )


def _sc_scatter(e, row2d):
    mesh = plsc.VectorSubcoreMesh(core_axis_name="c", subcore_axis_name="s")
    f = pl.kernel(
        _sc_scatter_body,
        out_type=jax.ShapeDtypeStruct((NC, NPAD, 16), jnp.float32),
        mesh=mesh,
        scratch_types=[
            pltpu.VMEM_SHARED((NPAD, 16), jnp.float32),
            pltpu.VMEM((CH, 16), jnp.float32),
            pltpu.VMEM((CH // 128, 128), jnp.int32),
        ],
        compiler_params=pltpu.CompilerParams(use_tc_tiling_on_sc=False),
    )
    return f(e, row2d)


def _node_mlp_body(p_ref, w3_ref, b3_ref, w4_ref, b4_ref, o_ref):
    x = p_ref[0] + p_ref[1]
    y = _silu(jnp.dot(x.astype(jnp.bfloat16), w3_ref[...].astype(jnp.bfloat16),
                      preferred_element_type=jnp.float32) + b3_ref[...])
    o_ref[...] = jnp.dot(y.astype(jnp.bfloat16), w4_ref[...].astype(jnp.bfloat16),
                         preferred_element_type=jnp.float32) + b4_ref[...]


def _node_mlp(p2, bd3, b3t, bd4, b4t):
    blk = 1600
    grid = (NPAD // 8) // blk
    return pl.pallas_call(
        _node_mlp_body,
        grid=(grid,),
        in_specs=[
            pl.BlockSpec((NC, blk, 128), lambda i: (0, i, 0)),
            pl.BlockSpec((128, 128), lambda i: (0, 0)),
            pl.BlockSpec((1, 128), lambda i: (0, 0)),
            pl.BlockSpec((128, 1024), lambda i: (0, 0)),
            pl.BlockSpec((1, 1024), lambda i: (0, 0)),
        ],
        out_specs=pl.BlockSpec((blk, 1024), lambda i: (i, 0)),
        out_shape=jax.ShapeDtypeStruct((NPAD // 8, 1024), jnp.float32),
    )(p2, bd3, b3t, bd4, b4t)


def kernel(edge_index, edge_attr, W1, b1, W2, b2, W3, b3, W4, b4):
    eye8 = jnp.eye(8, dtype=jnp.float32)
    bd1 = jnp.kron(eye8, W1)
    bd2 = jnp.kron(eye8, W2)
    bd3 = jnp.kron(eye8, W3)
    bd4 = jnp.kron(eye8, W4)
    b1t = jnp.tile(b1, 8)[None, :]
    b2t = jnp.tile(b2, 8)[None, :]
    b3t = jnp.tile(b3, 8)[None, :]
    b4t = jnp.tile(b4, 8)[None, :]

    xT = edge_attr.T
    e2 = _edge_mlp(xT, bd1, b1t, bd2, b2t)
    e = e2.reshape(E, 16)

    row = edge_index[0]
    row_perm = row.reshape(E // BLK, 8, G).swapaxes(1, 2).reshape(E)
    row2d = row_perm.reshape(E // 128, 128)

    p = _sc_scatter(e, row2d)
    p_packed = p.reshape(NC, NPAD // 8, 128)
    h_full = _node_mlp(p_packed, bd3, b3t, bd4, b4t)
    return h_full.reshape(NPAD, 128)[:N_NODES]


# speedup vs baseline: 1.9843x; 1.0299x over previous
import functools

import jax
import jax.numpy as jnp
from jax import lax
from jax.experimental import pallas as pl
from jax.experimental.pallas import tpu as pltpu
from jax.experimental.pallas import tpu_sc as plsc

E = 3200000
N_NODES = 100000
NPAD = 102400
NC, NS = 2, 16
NW = NC * NS
CH = 1024
K = 5
ES = E // K
NCHUNK_S = ES // CH
CHUNKS_PER_TILE = NCHUNK_S // NW + 1
BLK = 25600
GRID_S = ES // BLK
G = BLK // 8


def _silu(x):
    return x * jax.nn.sigmoid(x)


def _edge_mlp_body(x_ref, w1_ref, b1_ref, w2_ref, b2_ref, o_ref):
    xb = x_ref[...]
    xt = jnp.concatenate([xb[:, s * G:(s + 1) * G] for s in range(8)], axis=0)
    xt = xt.astype(jnp.bfloat16)
    w1 = w1_ref[...].astype(jnp.bfloat16)
    w2 = w2_ref[...].astype(jnp.bfloat16)
    y = lax.dot_general(xt, w1, (((0,), (0,)), ((), ())),
                        preferred_element_type=jnp.float32)
    y = _silu(y + b1_ref[...])
    z = jnp.dot(y.astype(jnp.bfloat16), w2,
                preferred_element_type=jnp.float32) + b2_ref[...]
    o_ref[...] = _silu(z)


def _edge_mlp(xT, bd1, b1t, bd2, b2t, s):
    return pl.pallas_call(
        _edge_mlp_body,
        grid=(GRID_S,),
        in_specs=[
            pl.BlockSpec((16, BLK), lambda i, s=s: (0, s * GRID_S + i)),
            pl.BlockSpec((128, 128), lambda i: (0, 0)),
            pl.BlockSpec((1, 128), lambda i: (0, 0)),
            pl.BlockSpec((128, 128), lambda i: (0, 0)),
            pl.BlockSpec((1, 128), lambda i: (0, 0)),
        ],
        out_specs=pl.BlockSpec((G, 128), lambda i: (i, 0)),
        out_shape=jax.ShapeDtypeStruct((ES // 8, 128), jnp.float32),
    )(xT, bd1, b1t, bd2, b2t)


def _sc_scatter_body(slice_id, e_hbm, row2d_hbm, out_hbm, agg_sh, ebuf, ibuf):
    c = lax.axis_index("c")
    s = lax.axis_index("s")
    w = s * NC + c

    def _zrow(i, carry):
        ebuf[i] = jnp.zeros((16,), jnp.float32)
        return carry

    lax.fori_loop(0, CH, _zrow, 0)
    base = s * (NPAD // NS)
    for z in range(6):
        pltpu.sync_copy(ebuf, agg_sh.at[pl.ds(base + z * CH, CH)])
    pltpu.sync_copy(ebuf.at[pl.ds(0, 256)], agg_sh.at[pl.ds(base + 6 * CH, 256)])
    plsc.subcore_barrier()

    def _do_chunk(j):
        start_edge = pl.multiple_of(j * CH, CH)
        pltpu.sync_copy(e_hbm.at[pl.ds(start_edge, CH)], ebuf)
        row_off = slice_id * (NCHUNK_S * (CH // 128)) + j * (CH // 128)
        pltpu.sync_copy(
            row2d_hbm.at[pl.ds(pl.multiple_of(row_off, CH // 128), CH // 128)],
            ibuf,
        )
        for sub in range(CH // 128):
            pltpu.sync_copy(
                ebuf.at[pl.ds(sub * 128, 128)],
                agg_sh.at[ibuf.at[sub]],
                add=True,
            )

    def _loop(k, carry):
        j = w + NW * k

        @pl.when(j < NCHUNK_S)
        def _():
            _do_chunk(j)

        return carry

    lax.fori_loop(0, CHUNKS_PER_TILE, _loop, 0)
    plsc.subcore_barrier()

    pltpu.sync_copy(
        agg_sh.at[pl.ds(base, NPAD // NS)],
        out_hbm.at[c].at[pl.ds(base, NPAD // NS)],
    )


def _sc_scatter(e, row2d, slice_id):
    mesh = plsc.VectorSubcoreMesh(core_axis_name="c", subcore_axis_name="s")
    f = pl.kernel(
        functools.partial(_sc_scatter_body, slice_id),
        out_type=jax.ShapeDtypeStruct((NC, NPAD, 16), jnp.float32),
        mesh=mesh,
        scratch_types=[
            pltpu.VMEM_SHARED((NPAD, 16), jnp.float32),
            pltpu.VMEM((CH, 16), jnp.float32),
            pltpu.VMEM((CH // 128, 128), jnp.int32),
        ],
        compiler_params=pltpu.CompilerParams(use_tc_tiling_on_sc=False),
    )
    return f(e, row2d)


def _node_mlp_body(p0, p1, p2, p3, p4, w3_ref, b3_ref, w4_ref, b4_ref, o_ref):
    x = (p0[0] + p0[1] + p1[0] + p1[1] + p2[0] + p2[1]
         + p3[0] + p3[1] + p4[0] + p4[1])
    y = _silu(jnp.dot(x.astype(jnp.bfloat16), w3_ref[...].astype(jnp.bfloat16),
                      preferred_element_type=jnp.float32) + b3_ref[...])
    o_ref[...] = jnp.dot(y.astype(jnp.bfloat16), w4_ref[...].astype(jnp.bfloat16),
                         preferred_element_type=jnp.float32) + b4_ref[...]


def _node_mlp(parts, bd3, b3t, bd4, b4t):
    blk = 1600
    grid = (NPAD // 8) // blk
    pspec = pl.BlockSpec((NC, blk, 128), lambda i: (0, i, 0))
    return pl.pallas_call(
        _node_mlp_body,
        grid=(grid,),
        in_specs=[pspec] * K + [
            pl.BlockSpec((128, 128), lambda i: (0, 0)),
            pl.BlockSpec((1, 128), lambda i: (0, 0)),
            pl.BlockSpec((128, 1024), lambda i: (0, 0)),
            pl.BlockSpec((1, 1024), lambda i: (0, 0)),
        ],
        out_specs=pl.BlockSpec((blk, 1024), lambda i: (i, 0)),
        out_shape=jax.ShapeDtypeStruct((NPAD // 8, 1024), jnp.float32),
    )(*parts, bd3, b3t, bd4, b4t)


def kernel(edge_index, edge_attr, W1, b1, W2, b2, W3, b3, W4, b4):
    eye8 = jnp.eye(8, dtype=jnp.float32)
    bd1 = jnp.kron(eye8, W1)
    bd2 = jnp.kron(eye8, W2)
    bd3 = jnp.kron(eye8, W3)
    bd4 = jnp.kron(eye8, W4)
    b1t = jnp.tile(b1, 8)[None, :]
    b2t = jnp.tile(b2, 8)[None, :]
    b3t = jnp.tile(b3, 8)[None, :]
    b4t = jnp.tile(b4, 8)[None, :]

    xT = edge_attr.T

    row = edge_index[0]
    row_perm = row.reshape(E // BLK, 8, G).swapaxes(1, 2).reshape(E)
    row2d = row_perm.reshape(E // 128, 128)

    parts = []
    for s in range(K):
        e2 = _edge_mlp(xT, bd1, b1t, bd2, b2t, s)
        e = e2.reshape(ES, 16)
        parts.append(_sc_scatter(e, row2d, s))

    parts_packed = [p.reshape(NC, NPAD // 8, 128) for p in parts]
    h_full = _node_mlp(parts_packed, bd3, b3t, bd4, b4t)
    return h_full.reshape(NPAD, 128)[:N_NODES]
